# Initial kernel scaffold; baseline (speedup 1.0000x reference)
import functools

import jax
import jax.numpy as jnp
from jax import lax
from jax.experimental import pallas as pl
from jax.experimental.pallas import tpu as pltpu
from jax.experimental.pallas import tpu_sc as plsc

VOCAB_SIZE = 1000000
EMBED_SIZE = 64
BATCH = 16384
HIST_LEN = 50

B = BATCH * HIST_LEN
NC, NS = 2, 16
NW = NC * NS
B_PER_W = B // NW
CHUNK = 512
NCHUNK = B_PER_W // CHUNK


def _make_kernel():
  mesh = plsc.VectorSubcoreMesh(core_axis_name="c", subcore_axis_name="s")

  @functools.partial(
      pl.kernel,
      mesh=mesh,
      out_type=jax.ShapeDtypeStruct((B, EMBED_SIZE), jnp.float32),
      scratch_types=[
          pltpu.VMEM((CHUNK,), jnp.int32),
          pltpu.VMEM((CHUNK, EMBED_SIZE), jnp.float32),
          pltpu.SemaphoreType.DMA,
      ],
  )
  def emb(idx_hbm, table_hbm, out_hbm, idx_v, rows_v, sem):
    wid = lax.axis_index("s") * NC + lax.axis_index("c")
    base = wid * B_PER_W

    def body(g, carry):
      off = base + g * CHUNK
      pltpu.sync_copy(idx_hbm.at[pl.ds(off, CHUNK)], idx_v)
      pltpu.async_copy(table_hbm.at[idx_v], rows_v, sem).wait()
      pltpu.sync_copy(rows_v, out_hbm.at[pl.ds(off, CHUNK)])
      return carry

    lax.fori_loop(0, NCHUNK, body, 0)

  return emb


_emb = _make_kernel()


@jax.jit
def kernel(x, table):
  idx = x.reshape(-1).astype(jnp.int32)
  out = _emb(idx, table)
  return out.reshape(BATCH, HIST_LEN, EMBED_SIZE)


# speedup vs baseline: 1.7958x; 1.7958x over previous
import functools

import jax
import jax.numpy as jnp
from jax import lax
from jax.experimental import pallas as pl
from jax.experimental.pallas import tpu as pltpu
from jax.experimental.pallas import tpu_sc as plsc

VOCAB_SIZE = 1000000
EMBED_SIZE = 64
BATCH = 16384
HIST_LEN = 50

B = BATCH * HIST_LEN
NC, NS = 2, 16
NW = NC * NS
B_PER_W = B // NW
CHUNK = 512
NCHUNK = B_PER_W // CHUNK


def _make_kernel():
  mesh = plsc.VectorSubcoreMesh(core_axis_name="c", subcore_axis_name="s")

  @functools.partial(
      pl.kernel,
      mesh=mesh,
      out_type=jax.ShapeDtypeStruct((B, EMBED_SIZE), jnp.float32),
      scratch_types=[
          pltpu.VMEM((CHUNK,), jnp.int32),
          pltpu.VMEM((CHUNK, EMBED_SIZE), jnp.float32),
          pltpu.SemaphoreType.DMA,
      ],
      compiler_params=pltpu.CompilerParams(use_tc_tiling_on_sc=False),
  )
  def emb(idx_hbm, table_hbm, out_hbm, idx_v, rows_v, sem):
    wid = lax.axis_index("s") * NC + lax.axis_index("c")
    base = wid * B_PER_W

    def body(g, carry):
      off = base + g * CHUNK
      pltpu.sync_copy(idx_hbm.at[pl.ds(off, CHUNK)], idx_v)
      pltpu.async_copy(table_hbm.at[idx_v], rows_v, sem).wait()
      pltpu.sync_copy(rows_v, out_hbm.at[pl.ds(off, CHUNK)])
      return carry

    lax.fori_loop(0, NCHUNK, body, 0)

  return emb


_emb = _make_kernel()


@jax.jit
def kernel(x, table):
  idx = x.reshape(-1).astype(jnp.int32)
  out = _emb(idx, table)
  return out.reshape(BATCH, HIST_LEN, EMBED_SIZE)


# speedup vs baseline: 1.8700x; 1.0413x over previous
import functools

import jax
import jax.numpy as jnp
from jax import lax
from jax.experimental import pallas as pl
from jax.experimental.pallas import tpu as pltpu
from jax.experimental.pallas import tpu_sc as plsc

VOCAB_SIZE = 1000000
EMBED_SIZE = 64
BATCH = 16384
HIST_LEN = 50

B = BATCH * HIST_LEN
NC, NS = 2, 16
NW = NC * NS
B_PER_W = B // NW
CHUNK = 256
NCHUNK = B_PER_W // CHUNK
NBUF = 4
NITER = NCHUNK // NBUF


def _make_kernel():
  mesh = plsc.VectorSubcoreMesh(core_axis_name="c", subcore_axis_name="s")

  @functools.partial(
      pl.kernel,
      mesh=mesh,
      out_type=jax.ShapeDtypeStruct((B, EMBED_SIZE), jnp.float32),
      scratch_types=[
          pltpu.VMEM((B_PER_W,), jnp.int32),
          pltpu.VMEM((NBUF, CHUNK, EMBED_SIZE), jnp.float32),
          pltpu.SemaphoreType.DMA((NBUF,)),
          pltpu.SemaphoreType.DMA((NBUF,)),
      ],
      compiler_params=pltpu.CompilerParams(use_tc_tiling_on_sc=False),
  )
  def emb(idx_hbm, table_hbm, out_hbm, idx_all, rows_v, gsem, osem):
    wid = lax.axis_index("s") * NC + lax.axis_index("c")
    base = wid * B_PER_W
    pltpu.sync_copy(idx_hbm.at[pl.ds(base, B_PER_W)], idx_all)

    def gather_copy(g, b):
      return pltpu.make_async_copy(
          table_hbm.at[idx_all.at[pl.ds(g * CHUNK, CHUNK)]],
          rows_v.at[b], gsem.at[b])

    def wb_copy(g, b):
      return pltpu.make_async_copy(
          rows_v.at[b], out_hbm.at[pl.ds(base + g * CHUNK, CHUNK)],
          osem.at[b])

    for b in range(NBUF):
      gather_copy(b, b).start()

    def round_fn(i, start_next):
      g0 = i * NBUF
      for b in range(NBUF):
        gather_copy(g0 + b, b).wait()
        wb_copy(g0 + b, b).start()
      for b in range(NBUF):
        wb_copy(g0 + b, b).wait()
        if start_next:
          gather_copy(g0 + NBUF + b, b).start()

    def body(i, carry):
      round_fn(i, True)
      return carry

    lax.fori_loop(0, NITER - 1, body, 0)
    round_fn(NITER - 1, False)

  return emb


_emb = _make_kernel()


@jax.jit
def kernel(x, table):
  idx = x.reshape(-1).astype(jnp.int32)
  out = _emb(idx, table)
  return out.reshape(BATCH, HIST_LEN, EMBED_SIZE)
